# baseline (speedup 1.0000x reference)
import functools

import jax
import jax.numpy as jnp
import numpy as np
from jax import lax
from jax.experimental import pallas as pl
from jax.experimental.pallas import tpu as pltpu
from jax.experimental.pallas import tpu_sc as plsc

_M = 512 * 512
_C = 96
_K = 1000
_KPAD = 1024
_NW = 32
_ROWS = (2 * _KPAD * _C) // (_NW * 128)


def _build_tables():
    skey = jax.random.key(42)
    ka, kb = jax.random.split(skey)
    tabs = []
    for k in (ka, kb):
        kk = k
        for _ in range(2):
            kk, sub = jax.random.split(kk)
            bits = np.asarray(jax.random.bits(sub, (_M,), dtype=jnp.uint32))
            tabs.append(np.argsort(bits, kind="stable").astype(np.int32))
    return tabs


_A0_NP, _B0_NP, _A1_NP, _B1_NP = _build_tables()


def _select_chain(A, B, n, S_lab):
    j = jnp.arange(_K)
    SA = jnp.cumsum((A < n).astype(jnp.int32))
    SB = jnp.cumsum((B < n).astype(jnp.int32))
    rounds = (n > 1).astype(jnp.int32) + (n > 1625).astype(jnp.int32)
    iB = jnp.clip(jnp.searchsorted(SB, j + 1, side="left"), 0, _M - 1)
    P = B[iB]
    qA = jnp.where(rounds == 2, P + 1, j + 1)
    iA = jnp.clip(jnp.searchsorted(SA, qA, side="left"), 0, _M - 1)
    F = jnp.where(rounds == 0, j, A[iA])
    iL = jnp.clip(jnp.searchsorted(S_lab, F + 1, side="left"), 0, _M - 1)
    return iL.astype(jnp.int32)


def _sc_gather_body(table_hbm, idx_hbm, out_hbm, idx_v, rows_v, sem):
    w = lax.axis_index("s") * 2 + lax.axis_index("c")
    pltpu.sync_copy(idx_hbm.at[w], idx_v)

    def body(r, carry):
        pltpu.async_copy(table_hbm.at[idx_v.at[r]], rows_v.at[r], sem).wait()
        return carry

    lax.fori_loop(0, _ROWS, body, 0)
    pltpu.sync_copy(rows_v, out_hbm.at[w])


@jax.jit
def _sc_gather(table, idx):
    mesh = plsc.VectorSubcoreMesh(core_axis_name="c", subcore_axis_name="s")
    fn = functools.partial(
        pl.kernel,
        mesh=mesh,
        out_type=jax.ShapeDtypeStruct((_NW, _ROWS, 128), jnp.float32),
        scratch_types=[
            pltpu.VMEM((_ROWS, 128), jnp.int32),
            pltpu.VMEM((_ROWS, 128), jnp.float32),
            pltpu.SemaphoreType.DMA,
        ],
    )(_sc_gather_body)
    return fn(table, idx)


def _tc_loss_body(g_ref, lab_ref, out_ref):
    lab = lab_ref[...]
    cnt0 = jnp.sum((lab == 0).astype(jnp.int32))
    cnt1 = _M - cnt0
    npair = jnp.minimum(jnp.minimum(_K, cnt0), cnt1)
    e0 = g_ref[0]
    e1 = g_ref[1]
    dot = jnp.sum(e0 * e1, axis=1)
    ss0 = jnp.sum(e0 * e0, axis=1)
    ss1 = jnp.sum(e1 * e1, axis=1)
    n0 = jnp.maximum(jnp.sqrt(ss0), 1e-8)
    n1 = jnp.maximum(jnp.sqrt(ss1), 1e-8)
    cos = dot / (n0 * n1)
    jj = lax.broadcasted_iota(jnp.int32, (_KPAD,), 0)
    total = jnp.sum(jnp.where(jj < npair, cos, 0.0))
    loss = 1.0 - total / npair.astype(jnp.float32)
    empty = (cnt0 == 0) | (cnt1 == 0)
    res = jnp.where(empty, jnp.float32(0.0), loss)
    out_ref[...] = jnp.full((1, 1), res, dtype=jnp.float32)


def _tc_loss(g3, lab2d):
    return pl.pallas_call(
        _tc_loss_body,
        out_shape=jax.ShapeDtypeStruct((1, 1), jnp.float32),
    )(g3, lab2d)


def kernel(embeddings, labels):
    emb_flat = embeddings.reshape(-1)
    lab = labels.reshape(-1).astype(jnp.int32)
    S0 = jnp.cumsum((lab == 0).astype(jnp.int32))
    cnt0 = S0[-1]
    cnt1 = _M - cnt0
    S1 = jnp.arange(1, _M + 1, dtype=jnp.int32) - S0
    A0 = jnp.asarray(_A0_NP)
    B0 = jnp.asarray(_B0_NP)
    A1 = jnp.asarray(_A1_NP)
    B1 = jnp.asarray(_B1_NP)
    s0 = _select_chain(A0, B0, cnt0, S0)
    s1 = _select_chain(A1, B1, cnt1, S1)
    s0p = jnp.zeros(_KPAD, jnp.int32).at[:_K].set(s0)
    s1p = jnp.zeros(_KPAD, jnp.int32).at[:_K].set(s1)
    coff = jnp.arange(_C, dtype=jnp.int32) * _M
    flat_idx = jnp.stack([s0p, s1p])[:, :, None] + coff[None, None, :]
    idx = flat_idx.reshape(_NW, _ROWS, 128)
    gathered = _sc_gather(emb_flat, idx)
    g3 = gathered.reshape(2, _KPAD, _C)
    lab2d = lab.reshape(_M // 128, 128)
    out = _tc_loss(g3, lab2d)
    return out.reshape(())


# speedup vs baseline: 5.4564x; 5.4564x over previous
import functools

import jax
import jax.numpy as jnp
import numpy as np
from jax import lax
from jax.experimental import pallas as pl
from jax.experimental.pallas import tpu as pltpu
from jax.experimental.pallas import tpu_sc as plsc

_M = 512 * 512
_C = 96
_K = 1000
_KPAD = 1024
_NW = 32
_ROWS = (2 * _KPAD * _C) // (_NW * 128)


def _build_tables():
    skey = jax.random.key(42)
    ka, kb = jax.random.split(skey)
    tabs = []
    for k in (ka, kb):
        kk = k
        for _ in range(2):
            kk, sub = jax.random.split(kk)
            bits = np.asarray(jax.random.bits(sub, (_M,), dtype=jnp.uint32))
            tabs.append(np.argsort(bits, kind="stable").astype(np.int32))
    return tabs


_A0_NP, _B0_NP, _A1_NP, _B1_NP = _build_tables()

_NB = _M // 128
_Mf = float(_M)
_HI = jax.lax.Precision.HIGHEST


def _rank_pos(q, SA, bpT):
    f32 = jnp.float32
    nfull = jnp.sum((bpT < q).astype(f32), axis=1, keepdims=True)
    blk = jnp.minimum(nfull, float(_NB - 1)).astype(jnp.int32)
    ohc = lax.broadcasted_iota(jnp.int32, (_KPAD, _NB), 1)
    oh = (ohc == blk).astype(f32)
    row = jnp.dot(oh, SA, precision=_HI)
    rowcnt = jnp.sum((row < q).astype(f32), axis=1, keepdims=True)
    return jnp.where(nfull > float(_NB - 1), _Mf, 128.0 * nfull + rowcnt)


def _gather_elem(t2d, pos):
    f32 = jnp.float32
    p = jnp.clip(pos, 0.0, _Mf - 1.0)
    r = jnp.floor(p * (1.0 / 128.0))
    l = (p - 128.0 * r).astype(jnp.int32)
    ohc = lax.broadcasted_iota(jnp.int32, (_KPAD, _NB), 1)
    oh = (ohc == r.astype(jnp.int32)).astype(f32)
    row = jnp.dot(oh, t2d, precision=_HI)
    lanes = lax.broadcasted_iota(jnp.int32, (_KPAD, 128), 1)
    return jnp.sum(jnp.where(lanes == l, row, 0.0), axis=1, keepdims=True)


def _tc_select_body(lab_ref, a0_ref, b0_ref, a1_ref, b1_ref, s0_ref, s1_ref):
    f32 = jnp.float32
    lab = lab_ref[...]
    x_l0 = (lab == 0).astype(f32)
    n0 = jnp.sum(x_l0)
    n1 = _Mf - n0

    iota_l = lax.broadcasted_iota(jnp.int32, (128, 128), 0)
    iota_c = lax.broadcasted_iota(jnp.int32, (128, 128), 1)
    triu = (iota_l <= iota_c).astype(f32)

    def lane_cumsum(v):
        sh = 1
        while sh < _NB:
            z = jnp.zeros((1, sh), f32)
            v = v + jnp.concatenate([z, v[:, : _NB - sh]], axis=1)
            sh *= 2
        return v

    def two_level(x):
        W = jnp.dot(x, triu, precision=_HI)
        s = W[:, 127:128]
        st = jnp.transpose(s, (1, 0))
        incl = lane_cumsum(st)
        EB = jnp.transpose(incl - st, (1, 0))
        SA = W + EB
        return SA, incl

    def chain(j, a2d, b2d, SAa, bpTa, SAb, bpTb, SAl, bpTl, n):
        r1 = n > 1.0
        r2 = n > 1625.0
        posB = _rank_pos(j + 1.0, SAb, bpTb)
        P = _gather_elem(b2d, posB)
        qA = jnp.where(r2, P + 1.0, j + 1.0)
        posA = _rank_pos(qA, SAa, bpTa)
        F = jnp.where(r1, _gather_elem(a2d, posA), j)
        posL = _rank_pos(F + 1.0, SAl, bpTl)
        return jnp.clip(posL, 0.0, _Mf - 1.0)

    a0 = a0_ref[...].astype(f32)
    b0 = b0_ref[...].astype(f32)
    a1 = a1_ref[...].astype(f32)
    b1 = b1_ref[...].astype(f32)
    SAl0, bpTl0 = two_level(x_l0)
    SAl1, bpTl1 = two_level(1.0 - x_l0)
    SAa0, bpTa0 = two_level((a0 < n0).astype(f32))
    SAb0, bpTb0 = two_level((b0 < n0).astype(f32))
    SAa1, bpTa1 = two_level((a1 < n1).astype(f32))
    SAb1, bpTb1 = two_level((b1 < n1).astype(f32))

    j = lax.broadcasted_iota(jnp.int32, (_KPAD, 1), 0).astype(f32)
    s0 = chain(j, a0, b0, SAa0, bpTa0, SAb0, bpTb0, SAl0, bpTl0, n0)
    s1 = chain(j, a1, b1, SAa1, bpTa1, SAb1, bpTb1, SAl1, bpTl1, n1)
    s0_ref[...] = s0.astype(jnp.int32)
    s1_ref[...] = s1.astype(jnp.int32)


def _tc_select(lab2d, a0, b0, a1, b1):
    return pl.pallas_call(
        _tc_select_body,
        out_shape=[
            jax.ShapeDtypeStruct((_KPAD, 1), jnp.int32),
            jax.ShapeDtypeStruct((_KPAD, 1), jnp.int32),
        ],
    )(lab2d, a0, b0, a1, b1)


def _sc_gather_body(table_hbm, idx_hbm, out_hbm, idx_v, rows_v, sem):
    w = lax.axis_index("s") * 2 + lax.axis_index("c")
    pltpu.sync_copy(idx_hbm.at[w], idx_v)

    def body(r, carry):
        pltpu.async_copy(table_hbm.at[idx_v.at[r]], rows_v.at[r], sem).wait()
        return carry

    lax.fori_loop(0, _ROWS, body, 0)
    pltpu.sync_copy(rows_v, out_hbm.at[w])


@jax.jit
def _sc_gather(table, idx):
    mesh = plsc.VectorSubcoreMesh(core_axis_name="c", subcore_axis_name="s")
    fn = functools.partial(
        pl.kernel,
        mesh=mesh,
        out_type=jax.ShapeDtypeStruct((_NW, _ROWS, 128), jnp.float32),
        scratch_types=[
            pltpu.VMEM((_ROWS, 128), jnp.int32),
            pltpu.VMEM((_ROWS, 128), jnp.float32),
            pltpu.SemaphoreType.DMA,
        ],
    )(_sc_gather_body)
    return fn(table, idx)


def _tc_loss_body(g_ref, lab_ref, out_ref):
    lab = lab_ref[...]
    cnt0 = jnp.sum((lab == 0).astype(jnp.int32))
    cnt1 = _M - cnt0
    npair = jnp.minimum(jnp.minimum(_K, cnt0), cnt1)
    e0 = g_ref[0]
    e1 = g_ref[1]
    dot = jnp.sum(e0 * e1, axis=1)
    ss0 = jnp.sum(e0 * e0, axis=1)
    ss1 = jnp.sum(e1 * e1, axis=1)
    n0 = jnp.maximum(jnp.sqrt(ss0), 1e-8)
    n1 = jnp.maximum(jnp.sqrt(ss1), 1e-8)
    cos = dot / (n0 * n1)
    jj = lax.broadcasted_iota(jnp.int32, (_KPAD,), 0)
    total = jnp.sum(jnp.where(jj < npair, cos, 0.0))
    loss = 1.0 - total / npair.astype(jnp.float32)
    empty = (cnt0 == 0) | (cnt1 == 0)
    res = jnp.where(empty, jnp.float32(0.0), loss)
    out_ref[...] = jnp.full((1, 1), res, dtype=jnp.float32)


def _tc_loss(g3, lab2d):
    return pl.pallas_call(
        _tc_loss_body,
        out_shape=jax.ShapeDtypeStruct((1, 1), jnp.float32),
    )(g3, lab2d)


def kernel(embeddings, labels):
    emb_flat = embeddings.reshape(-1)
    lab2d = labels.reshape(_NB, 128).astype(jnp.int32)
    A0 = jnp.asarray(_A0_NP.reshape(_NB, 128))
    B0 = jnp.asarray(_B0_NP.reshape(_NB, 128))
    A1 = jnp.asarray(_A1_NP.reshape(_NB, 128))
    B1 = jnp.asarray(_B1_NP.reshape(_NB, 128))
    s0, s1 = _tc_select(lab2d, A0, B0, A1, B1)
    coff = jnp.arange(_C, dtype=jnp.int32) * _M
    flat_idx = jnp.stack([s0.reshape(_KPAD), s1.reshape(_KPAD)])[:, :, None] \
        + coff[None, None, :]
    idx = flat_idx.reshape(_NW, _ROWS, 128)
    gathered = _sc_gather(emb_flat, idx)
    g3 = gathered.reshape(2, _KPAD, _C)
    out = _tc_loss(g3, lab2d)
    return out.reshape(())
